# baseline (speedup 1.0000x reference)
import functools

import jax
import jax.numpy as jnp
from jax import lax
from jax.experimental import pallas as pl
from jax.experimental.pallas import tpu as pltpu
from jax.experimental.pallas import tpu_sc as plsc

TM = 256
NT_MAX = 24
NTOT = NT_MAX * TM
NC, NS = 2, 16
NW = NC * NS


def _gating_kernel(x_ref, wg_ref, g_ref):
    x = x_ref[...]
    wg = wg_ref[...]
    s = jax.lax.dot_general(x, wg, (((1,), (1,)), ((), ())),
                            preferred_element_type=jnp.float32)
    E = s.shape[-1]
    ids = jax.lax.broadcasted_iota(jnp.int32, s.shape, 1)
    m1 = jnp.max(s, axis=-1, keepdims=True)
    i1 = jnp.min(jnp.where(s == m1, ids, E), axis=-1, keepdims=True)
    s_masked = jnp.where(ids == i1, -jnp.inf, s)
    m2 = jnp.max(s_masked, axis=-1, keepdims=True)
    i2 = jnp.min(jnp.where(s_masked == m2, ids, E), axis=-1, keepdims=True)
    p1 = jax.nn.sigmoid(m1 - m2)
    p2 = jnp.maximum(1.0 - p1, jnp.float32(1e-30))
    g_ref[...] = jnp.where(ids == i1, p1, 0.0) + jnp.where(ids == i2, p2, 0.0)


def _routing_metadata(g):
    T, E = g.shape
    onehot = g > 0
    oh = onehot.astype(jnp.int32)
    counts = jnp.sum(oh, axis=0)
    ntiles = (counts + TM - 1) // TM
    cum_tiles = jnp.cumsum(ntiles)
    pad_start = ((cum_tiles - ntiles) * TM).astype(jnp.int32)
    ranks = jnp.cumsum(oh, axis=0) - oh
    dst = pad_start[None, :] + ranks
    n_active = cum_tiles[-1].astype(jnp.int32)
    te = jnp.searchsorted(cum_tiles, jnp.arange(NT_MAX, dtype=jnp.int32),
                          side="right").astype(jnp.int32)
    te_last = te[jnp.maximum(n_active - 1, 0)]
    te = jnp.where(jnp.arange(NT_MAX) < n_active, te, te_last)
    meta = jnp.concatenate([te, n_active[None]])
    q0 = jnp.min(jnp.where(onehot, dst, NTOT), axis=1).astype(jnp.int32)
    q1 = jnp.max(jnp.where(onehot, dst, -1), axis=1).astype(jnp.int32)
    ids = jax.lax.broadcasted_iota(jnp.int32, (T, E), 1)
    e_lo = jnp.min(jnp.where(onehot, ids, E), axis=1)
    e_hi = jnp.max(jnp.where(onehot, ids, -1), axis=1)
    w_lo = jnp.take_along_axis(g, e_lo[:, None], axis=1)[:, 0]
    w_hi = jnp.take_along_axis(g, e_hi[:, None], axis=1)[:, 0]
    p0_128 = jnp.broadcast_to(w_lo[:, None], (T, 128))
    p1_128 = jnp.broadcast_to(w_hi[:, None], (T, 128))
    return meta, q0, q1, p0_128, p1_128


TPW = 2048 // NW


@functools.partial(
    pl.kernel,
    mesh=plsc.VectorSubcoreMesh(core_axis_name="c", subcore_axis_name="s"),
    out_type=(
        jax.ShapeDtypeStruct((NTOT, 1024), jnp.float32),
        jax.ShapeDtypeStruct((NTOT, 128), jnp.float32),
    ),
    scratch_types=[
        pltpu.VMEM((TPW, 1024), jnp.float32),
        pltpu.VMEM((TPW, 128), jnp.float32),
        pltpu.VMEM((TPW, 128), jnp.float32),
        pltpu.VMEM((TPW,), jnp.int32),
        pltpu.VMEM((TPW,), jnp.int32),
        pltpu.SemaphoreType.DMA,
    ],
)
def _dispatch_sc(x_hbm, q0_hbm, q1_hbm, p0_hbm, p1_hbm, xs_hbm, wv_hbm,
                 rows_v, w0_v, w1_v, i0_v, i1_v, sem):
    wid = lax.axis_index("s") * NC + lax.axis_index("c")
    base = wid * TPW
    pltpu.sync_copy(x_hbm.at[pl.ds(base, TPW)], rows_v)
    pltpu.sync_copy(p0_hbm.at[pl.ds(base, TPW)], w0_v)
    pltpu.sync_copy(p1_hbm.at[pl.ds(base, TPW)], w1_v)
    pltpu.sync_copy(q0_hbm.at[pl.ds(base, TPW)], i0_v)
    pltpu.sync_copy(q1_hbm.at[pl.ds(base, TPW)], i1_v)
    c0 = pltpu.async_copy(rows_v, xs_hbm.at[i0_v], sem)
    c1 = pltpu.async_copy(rows_v, xs_hbm.at[i1_v], sem)
    c2 = pltpu.async_copy(w0_v, wv_hbm.at[i0_v], sem)
    c3 = pltpu.async_copy(w1_v, wv_hbm.at[i1_v], sem)
    c0.wait()
    c1.wait()
    c2.wait()
    c3.wait()


CH = 32


@functools.partial(
    pl.kernel,
    mesh=plsc.VectorSubcoreMesh(core_axis_name="c", subcore_axis_name="s"),
    out_type=jax.ShapeDtypeStruct((2048, 1024), jnp.float32),
    scratch_types=[
        pltpu.VMEM((CH, 1024), jnp.float32),
        pltpu.VMEM((CH, 1024), jnp.float32),
        pltpu.VMEM((CH,), jnp.int32),
        pltpu.VMEM((CH,), jnp.int32),
        pltpu.SemaphoreType.DMA,
    ],
)
def _combine_sc(ys_hbm, q0_hbm, q1_hbm, out_hbm, a_v, b_v, i0_v, i1_v, sem):
    wid = lax.axis_index("s") * NC + lax.axis_index("c")
    for c in range(TPW // CH):
        base = wid * TPW + c * CH
        pltpu.sync_copy(q0_hbm.at[pl.ds(base, CH)], i0_v)
        pltpu.sync_copy(q1_hbm.at[pl.ds(base, CH)], i1_v)
        g0 = pltpu.async_copy(ys_hbm.at[i0_v], a_v, sem)
        g1 = pltpu.async_copy(ys_hbm.at[i1_v], b_v, sem)
        g0.wait()
        g1.wait()

        def row_body(r, _):
            def col_body(j, _):
                va = a_v[r, pl.ds(j * 16, 16)]
                vb = b_v[r, pl.ds(j * 16, 16)]
                a_v[r, pl.ds(j * 16, 16)] = va + vb
                return 0
            return lax.fori_loop(0, 1024 // 16, col_body, 0, unroll=8)

        lax.fori_loop(0, CH, row_body, 0)
        pltpu.sync_copy(a_v, out_hbm.at[pl.ds(base, CH)])


def _ffn_kernel(meta_ref, xs_ref, wv_ref, w1_ref, w2_ref, w3_ref, ys_ref):
    j = pl.program_id(0)
    n_active = meta_ref[NT_MAX]

    @pl.when(j < n_active)
    def _():
        xb = xs_ref[...]
        w1 = w1_ref[0]
        w2 = w2_ref[0]
        w3 = w3_ref[0]
        h1 = jnp.dot(xb, w1, preferred_element_type=jnp.float32)
        h2 = jnp.dot(xb, w2, preferred_element_type=jnp.float32)
        h = (h1 * jax.nn.sigmoid(h1)) * h2
        y = jnp.dot(h, w3, preferred_element_type=jnp.float32)
        ys_ref[...] = y * wv_ref[:, 0:1]


def kernel(x, Wg, W1, W2, W3):
    B, T, D = x.shape
    E, _, F = W1.shape
    x2 = x.reshape(T, D)

    g = pl.pallas_call(
        _gating_kernel,
        out_shape=jax.ShapeDtypeStruct((T, E), jnp.float32),
    )(x2, Wg)

    meta, q0, q1, p0_128, p1_128 = _routing_metadata(g)

    xs, wv = _dispatch_sc(x2, q0, q1, p0_128, p1_128)

    ys = pl.pallas_call(
        _ffn_kernel,
        grid_spec=pltpu.PrefetchScalarGridSpec(
            num_scalar_prefetch=1,
            grid=(NT_MAX,),
            in_specs=[
                pl.BlockSpec((TM, D), lambda j, m: (j, 0)),
                pl.BlockSpec((TM, 128), lambda j, m: (j, 0)),
                pl.BlockSpec((1, D, F), lambda j, m: (m[j], 0, 0)),
                pl.BlockSpec((1, D, F), lambda j, m: (m[j], 0, 0)),
                pl.BlockSpec((1, F, D), lambda j, m: (m[j], 0, 0)),
            ],
            out_specs=pl.BlockSpec((TM, D), lambda j, m: (j, 0)),
        ),
        out_shape=jax.ShapeDtypeStruct((NTOT, D), jnp.float32),
        compiler_params=pltpu.CompilerParams(
            dimension_semantics=("arbitrary",),
        ),
    )(meta, xs, wv, W1, W2, W3)

    out = _combine_sc(ys, q0, q1)

    return out.reshape(B, T, D)


# speedup vs baseline: 1.0155x; 1.0155x over previous
import functools

import jax
import jax.numpy as jnp
from jax import lax
from jax.experimental import pallas as pl
from jax.experimental.pallas import tpu as pltpu
from jax.experimental.pallas import tpu_sc as plsc

TM = 256
NT_MAX = 24
NTOT = NT_MAX * TM
NC, NS = 2, 16
NW = NC * NS


def _gating_kernel(x_ref, wg_ref, g_ref):
    x = x_ref[...]
    wg = wg_ref[...]
    s = jax.lax.dot_general(x, wg, (((1,), (1,)), ((), ())),
                            preferred_element_type=jnp.float32)
    E = s.shape[-1]
    ids = jax.lax.broadcasted_iota(jnp.int32, s.shape, 1)
    m1 = jnp.max(s, axis=-1, keepdims=True)
    i1 = jnp.min(jnp.where(s == m1, ids, E), axis=-1, keepdims=True)
    s_masked = jnp.where(ids == i1, -jnp.inf, s)
    m2 = jnp.max(s_masked, axis=-1, keepdims=True)
    i2 = jnp.min(jnp.where(s_masked == m2, ids, E), axis=-1, keepdims=True)
    p1 = jax.nn.sigmoid(m1 - m2)
    p2 = jnp.maximum(1.0 - p1, jnp.float32(1e-30))
    g_ref[...] = jnp.where(ids == i1, p1, 0.0) + jnp.where(ids == i2, p2, 0.0)


def _routing_metadata(g):
    T, E = g.shape
    onehot = g > 0
    oh = onehot.astype(jnp.int32)
    counts = jnp.sum(oh, axis=0)
    ntiles = (counts + TM - 1) // TM
    cum_tiles = jnp.cumsum(ntiles)
    pad_start = ((cum_tiles - ntiles) * TM).astype(jnp.int32)
    ranks = jnp.cumsum(oh, axis=0) - oh
    dst = pad_start[None, :] + ranks
    n_active = cum_tiles[-1].astype(jnp.int32)
    te = jnp.searchsorted(cum_tiles, jnp.arange(NT_MAX, dtype=jnp.int32),
                          side="right").astype(jnp.int32)
    te_last = te[jnp.maximum(n_active - 1, 0)]
    te = jnp.where(jnp.arange(NT_MAX) < n_active, te, te_last)
    meta = jnp.concatenate([te, n_active[None]])
    q0 = jnp.min(jnp.where(onehot, dst, NTOT), axis=1).astype(jnp.int32)
    q1 = jnp.max(jnp.where(onehot, dst, -1), axis=1).astype(jnp.int32)
    ids = jax.lax.broadcasted_iota(jnp.int32, (T, E), 1)
    e_lo = jnp.min(jnp.where(onehot, ids, E), axis=1)
    e_hi = jnp.max(jnp.where(onehot, ids, -1), axis=1)
    w_lo = jnp.take_along_axis(g, e_lo[:, None], axis=1)[:, 0]
    w_hi = jnp.take_along_axis(g, e_hi[:, None], axis=1)[:, 0]
    p0_128 = jnp.broadcast_to(w_lo[:, None], (T, 128))
    p1_128 = jnp.broadcast_to(w_hi[:, None], (T, 128))
    return meta, q0, q1, p0_128, p1_128


TPW = 2048 // NW


@functools.partial(
    pl.kernel,
    mesh=plsc.VectorSubcoreMesh(core_axis_name="c", subcore_axis_name="s"),
    out_type=(
        jax.ShapeDtypeStruct((NTOT, 1024), jnp.float32),
        jax.ShapeDtypeStruct((NTOT, 128), jnp.float32),
    ),
    scratch_types=[
        pltpu.VMEM((TPW, 1024), jnp.float32),
        pltpu.VMEM((TPW, 128), jnp.float32),
        pltpu.VMEM((TPW, 128), jnp.float32),
        pltpu.VMEM((TPW,), jnp.int32),
        pltpu.VMEM((TPW,), jnp.int32),
        pltpu.SemaphoreType.DMA,
    ],
)
def _dispatch_sc(x_hbm, q0_hbm, q1_hbm, p0_hbm, p1_hbm, xs_hbm, wv_hbm,
                 rows_v, w0_v, w1_v, i0_v, i1_v, sem):
    wid = lax.axis_index("s") * NC + lax.axis_index("c")
    base = wid * TPW
    pltpu.sync_copy(x_hbm.at[pl.ds(base, TPW)], rows_v)
    pltpu.sync_copy(p0_hbm.at[pl.ds(base, TPW)], w0_v)
    pltpu.sync_copy(p1_hbm.at[pl.ds(base, TPW)], w1_v)
    pltpu.sync_copy(q0_hbm.at[pl.ds(base, TPW)], i0_v)
    pltpu.sync_copy(q1_hbm.at[pl.ds(base, TPW)], i1_v)
    c0 = pltpu.async_copy(rows_v, xs_hbm.at[i0_v], sem)
    c1 = pltpu.async_copy(rows_v, xs_hbm.at[i1_v], sem)
    c2 = pltpu.async_copy(w0_v, wv_hbm.at[i0_v], sem)
    c3 = pltpu.async_copy(w1_v, wv_hbm.at[i1_v], sem)
    c0.wait()
    c1.wait()
    c2.wait()
    c3.wait()


CH = 16


@functools.partial(
    pl.kernel,
    mesh=plsc.VectorSubcoreMesh(core_axis_name="c", subcore_axis_name="s"),
    out_type=jax.ShapeDtypeStruct((2048, 1024), jnp.float32),
    scratch_types=[
        pltpu.VMEM((CH, 1024), jnp.float32),
        pltpu.VMEM((CH, 1024), jnp.float32),
        pltpu.VMEM((CH, 1024), jnp.float32),
        pltpu.VMEM((CH, 1024), jnp.float32),
        pltpu.VMEM((CH,), jnp.int32),
        pltpu.VMEM((CH,), jnp.int32),
        pltpu.VMEM((CH,), jnp.int32),
        pltpu.VMEM((CH,), jnp.int32),
        pltpu.SemaphoreType.DMA,
        pltpu.SemaphoreType.DMA,
    ],
)
def _combine_sc(ys_hbm, q0_hbm, q1_hbm, out_hbm,
                a0_v, b0_v, a1_v, b1_v, i00_v, i10_v, i01_v, i11_v,
                sem0, sem1):
    wid = lax.axis_index("s") * NC + lax.axis_index("c")
    bufs = ((a0_v, b0_v, i00_v, i10_v, sem0),
            (a1_v, b1_v, i01_v, i11_v, sem1))
    nch = TPW // CH

    def start(c):
        a_v, b_v, i0_v, i1_v, sem = bufs[c % 2]
        base = wid * TPW + c * CH
        pltpu.sync_copy(q0_hbm.at[pl.ds(base, CH)], i0_v)
        pltpu.sync_copy(q1_hbm.at[pl.ds(base, CH)], i1_v)
        g0 = pltpu.async_copy(ys_hbm.at[i0_v], a_v, sem)
        g1 = pltpu.async_copy(ys_hbm.at[i1_v], b_v, sem)
        return g0, g1

    handles = [start(0), None]
    for c in range(nch):
        if c + 1 < nch:
            handles[(c + 1) % 2] = start(c + 1)
        g0, g1 = handles[c % 2]
        g0.wait()
        g1.wait()
        a_v, b_v, _, _, _ = bufs[c % 2]

        def row_body(r, _):
            def col_body(j, _):
                va = a_v[r, pl.ds(j * 16, 16)]
                vb = b_v[r, pl.ds(j * 16, 16)]
                a_v[r, pl.ds(j * 16, 16)] = va + vb
                return 0
            return lax.fori_loop(0, 1024 // 16, col_body, 0, unroll=8)

        lax.fori_loop(0, CH, row_body, 0)
        pltpu.sync_copy(a_v, out_hbm.at[pl.ds(wid * TPW + c * CH, CH)])


def _ffn_kernel(meta_ref, xs_ref, wv_ref, w1_ref, w2_ref, w3_ref, ys_ref):
    j = pl.program_id(0)
    n_active = meta_ref[NT_MAX]

    @pl.when(j < n_active)
    def _():
        xb = xs_ref[...]
        w1 = w1_ref[0]
        w2 = w2_ref[0]
        w3 = w3_ref[0]
        h1 = jnp.dot(xb, w1, preferred_element_type=jnp.float32)
        h2 = jnp.dot(xb, w2, preferred_element_type=jnp.float32)
        h = (h1 * jax.nn.sigmoid(h1)) * h2
        y = jnp.dot(h, w3, preferred_element_type=jnp.float32)
        ys_ref[...] = y * wv_ref[:, 0:1]


def kernel(x, Wg, W1, W2, W3):
    B, T, D = x.shape
    E, _, F = W1.shape
    x2 = x.reshape(T, D)

    g = pl.pallas_call(
        _gating_kernel,
        out_shape=jax.ShapeDtypeStruct((T, E), jnp.float32),
    )(x2, Wg)

    meta, q0, q1, p0_128, p1_128 = _routing_metadata(g)

    xs, wv = _dispatch_sc(x2, q0, q1, p0_128, p1_128)

    ys = pl.pallas_call(
        _ffn_kernel,
        grid_spec=pltpu.PrefetchScalarGridSpec(
            num_scalar_prefetch=1,
            grid=(NT_MAX,),
            in_specs=[
                pl.BlockSpec((TM, D), lambda j, m: (j, 0)),
                pl.BlockSpec((TM, 128), lambda j, m: (j, 0)),
                pl.BlockSpec((1, D, F), lambda j, m: (m[j], 0, 0)),
                pl.BlockSpec((1, D, F), lambda j, m: (m[j], 0, 0)),
                pl.BlockSpec((1, F, D), lambda j, m: (m[j], 0, 0)),
            ],
            out_specs=pl.BlockSpec((TM, D), lambda j, m: (j, 0)),
        ),
        out_shape=jax.ShapeDtypeStruct((NTOT, D), jnp.float32),
        compiler_params=pltpu.CompilerParams(
            dimension_semantics=("arbitrary",),
        ),
    )(meta, xs, wv, W1, W2, W3)

    out = _combine_sc(ys, q0, q1)

    return out.reshape(B, T, D)
